# Initial kernel scaffold; baseline (speedup 1.0000x reference)
import jax
import jax.numpy as jnp
from jax.experimental import pallas as pl


def kernel(inputs, table):
    raise NotImplementedError("write your pallas kernel here")


# speedup vs baseline: 4.8095x; 4.8095x over previous
import functools

import jax
import jax.numpy as jnp
from jax import lax
from jax.experimental import pallas as pl
from jax.experimental.pallas import tpu as pltpu
from jax.experimental.pallas import tpu_sc as plsc

LANE = 128
KPG = 8
G = KPG * LANE


@functools.lru_cache(maxsize=None)
def _build(n_rows128, vocab, dim):
    mesh = plsc.VectorSubcoreMesh(core_axis_name="c", subcore_axis_name="s")
    nw = mesh.num_cores * mesh.num_subcores
    assert n_rows128 % (nw * KPG) == 0
    rows_per_worker = n_rows128 // nw
    groups = rows_per_worker // KPG
    n_idx = n_rows128 * LANE

    @functools.partial(
        pl.kernel,
        mesh=mesh,
        out_type=jax.ShapeDtypeStruct((n_idx, dim), jnp.float32),
        scratch_types=[
            pltpu.VMEM((KPG, LANE), jnp.int32),
            pltpu.VMEM((G, dim), jnp.float32),
            pltpu.SemaphoreType.DMA,
        ],
        compiler_params=pltpu.CompilerParams(use_tc_tiling_on_sc=False),
    )
    def k(idx_hbm, table_hbm, out_hbm, idx_v, rows_v, gsem):
        wid = lax.axis_index("s") * mesh.num_cores + lax.axis_index("c")
        row0 = wid * rows_per_worker

        def body(g, carry):
            r = row0 + g * KPG
            pltpu.sync_copy(idx_hbm.at[pl.ds(r, KPG)], idx_v)
            copies = [
                pltpu.async_copy(
                    table_hbm.at[idx_v.at[j]],
                    rows_v.at[pl.ds(j * LANE, LANE)],
                    gsem,
                )
                for j in range(KPG)
            ]
            for c in copies:
                c.wait()
            pltpu.sync_copy(rows_v, out_hbm.at[pl.ds(r * LANE, G)])
            return carry

        lax.fori_loop(0, groups, body, 0)

    return k


def kernel(inputs, table):
    b, h = inputs.shape
    vocab, dim = table.shape
    idx2d = inputs.astype(jnp.int32).reshape(-1, LANE)
    out = _build(idx2d.shape[0], vocab, dim)(idx2d, table)
    return out.reshape(b, h, dim)


# speedup vs baseline: 5.0310x; 1.0461x over previous
import functools

import jax
import jax.numpy as jnp
from jax import lax
from jax.experimental import pallas as pl
from jax.experimental.pallas import tpu as pltpu
from jax.experimental.pallas import tpu_sc as plsc

LANE = 128
KPG = 8
G = KPG * LANE
NBUF = 2


@functools.lru_cache(maxsize=None)
def _build(n_rows128, vocab, dim):
    mesh = plsc.VectorSubcoreMesh(core_axis_name="c", subcore_axis_name="s")
    nw = mesh.num_cores * mesh.num_subcores
    assert n_rows128 % (nw * KPG) == 0
    rows_per_worker = n_rows128 // nw
    groups = rows_per_worker // KPG
    assert groups % NBUF == 0 and groups >= 2 * NBUF
    n_idx = n_rows128 * LANE

    @functools.partial(
        pl.kernel,
        mesh=mesh,
        out_type=jax.ShapeDtypeStruct((n_idx, dim), jnp.float32),
        scratch_types=[
            pltpu.VMEM((NBUF, KPG, LANE), jnp.int32),
            pltpu.VMEM((NBUF, G, dim), jnp.float32),
            [pltpu.SemaphoreType.DMA] * NBUF,
            [pltpu.SemaphoreType.DMA] * NBUF,
            [pltpu.SemaphoreType.DMA] * NBUF,
        ],
        compiler_params=pltpu.CompilerParams(use_tc_tiling_on_sc=False),
    )
    def k(idx_hbm, table_hbm, out_hbm, idx_v, rows_v, isem, gsem, osem):
        wid = lax.axis_index("s") * mesh.num_cores + lax.axis_index("c")
        row0 = wid * rows_per_worker

        def fire_idx(g, b):
            pltpu.async_copy(
                idx_hbm.at[pl.ds(row0 + g * KPG, KPG)], idx_v.at[b], isem[b]
            )

        def drain_idx(b):
            pltpu.make_async_copy(
                idx_hbm.at[pl.ds(0, KPG)], idx_v.at[b], isem[b]
            ).wait()

        def fire_gathers(g, b):
            for j in range(KPG):
                pltpu.async_copy(
                    table_hbm.at[idx_v.at[b, j]],
                    rows_v.at[b, pl.ds(j * LANE, LANE)],
                    gsem[b],
                )

        def drain_gathers(b):
            for j in range(KPG):
                pltpu.make_async_copy(
                    out_hbm.at[pl.ds(0, LANE)],
                    rows_v.at[b, pl.ds(j * LANE, LANE)],
                    gsem[b],
                ).wait()

        def fire_out(g, b):
            pltpu.async_copy(
                rows_v.at[b], out_hbm.at[pl.ds((row0 + g * KPG) * LANE, G)],
                osem[b],
            )

        def drain_out(b):
            pltpu.make_async_copy(
                rows_v.at[b], out_hbm.at[pl.ds(0, G)], osem[b]
            ).wait()

        fire_idx(0, 0)
        fire_idx(1, 1)
        drain_idx(0)
        fire_gathers(0, 0)
        drain_gathers(0)
        fire_out(0, 0)
        fire_idx(2, 0)
        drain_idx(1)
        fire_gathers(1, 1)

        def step(s, carry):
            t = 2 * s + 2
            drain_gathers(1)
            fire_out(t - 1, 1)
            fire_idx(t + 1, 1)
            drain_out(0)
            drain_idx(0)
            fire_gathers(t, 0)
            drain_gathers(0)
            fire_out(t, 0)
            fire_idx(t + 2, 0)
            drain_out(1)
            drain_idx(1)
            fire_gathers(t + 1, 1)
            return carry

        lax.fori_loop(0, (groups - 4) // 2, step, 0)

        tg = groups - 2
        drain_gathers(1)
        fire_out(tg - 1, 1)
        fire_idx(tg + 1, 1)
        drain_out(0)
        drain_idx(0)
        fire_gathers(tg, 0)

        drain_gathers(0)
        fire_out(tg, 0)
        drain_out(1)
        drain_idx(1)
        fire_gathers(tg + 1, 1)

        drain_gathers(1)
        fire_out(tg + 1, 1)
        drain_out(0)
        drain_out(1)

    return k


def kernel(inputs, table):
    b, h = inputs.shape
    vocab, dim = table.shape
    idx2d = inputs.astype(jnp.int32).reshape(-1, LANE)
    out = _build(idx2d.shape[0], vocab, dim)(idx2d, table)
    return out.reshape(b, h, dim)


# speedup vs baseline: 5.0516x; 1.0041x over previous
import functools

import jax
import jax.numpy as jnp
from jax import lax
from jax.experimental import pallas as pl
from jax.experimental.pallas import tpu as pltpu
from jax.experimental.pallas import tpu_sc as plsc

LANE = 128
KPG = 8
G = KPG * LANE
NBUF = 3


@functools.lru_cache(maxsize=None)
def _build(n_rows128, vocab, dim):
    mesh = plsc.VectorSubcoreMesh(core_axis_name="c", subcore_axis_name="s")
    nw = mesh.num_cores * mesh.num_subcores
    assert n_rows128 % (nw * KPG) == 0
    rows_per_worker = n_rows128 // nw
    groups = rows_per_worker // KPG
    assert (groups - 4) % NBUF == 0 and groups >= 7
    n_idx = n_rows128 * LANE

    @functools.partial(
        pl.kernel,
        mesh=mesh,
        out_type=jax.ShapeDtypeStruct((n_idx, dim), jnp.float32),
        scratch_types=[
            pltpu.VMEM((NBUF, KPG, LANE), jnp.int32),
            pltpu.VMEM((NBUF, G, dim), jnp.float32),
            [pltpu.SemaphoreType.DMA] * NBUF,
            [pltpu.SemaphoreType.DMA] * NBUF,
            [pltpu.SemaphoreType.DMA] * NBUF,
        ],
        compiler_params=pltpu.CompilerParams(use_tc_tiling_on_sc=False),
    )
    def k(idx_hbm, table_hbm, out_hbm, idx_v, rows_v, isem, gsem, osem):
        wid = lax.axis_index("s") * mesh.num_cores + lax.axis_index("c")
        row0 = wid * rows_per_worker

        def fire_idx(g, b):
            pltpu.async_copy(
                idx_hbm.at[pl.ds(row0 + g * KPG, KPG)], idx_v.at[b], isem[b]
            )

        def drain_idx(b):
            pltpu.make_async_copy(
                idx_hbm.at[pl.ds(0, KPG)], idx_v.at[b], isem[b]
            ).wait()

        def fire_gathers(g, b):
            for j in range(KPG):
                pltpu.async_copy(
                    table_hbm.at[idx_v.at[b, j]],
                    rows_v.at[b, pl.ds(j * LANE, LANE)],
                    gsem[b],
                )

        def drain_gathers(b):
            for j in range(KPG):
                pltpu.make_async_copy(
                    out_hbm.at[pl.ds(0, LANE)],
                    rows_v.at[b, pl.ds(j * LANE, LANE)],
                    gsem[b],
                ).wait()

        def fire_out(g, b):
            pltpu.async_copy(
                rows_v.at[b], out_hbm.at[pl.ds((row0 + g * KPG) * LANE, G)],
                osem[b],
            )

        def drain_out(b):
            pltpu.make_async_copy(
                rows_v.at[b], out_hbm.at[pl.ds(0, G)], osem[b]
            ).wait()

        def steady(u, b, guard_idx):
            bm2 = (b + 1) % NBUF
            drain_gathers(bm2)
            fire_out(u - 2, bm2)
            if guard_idx:
                @pl.when(u + 1 < groups)
                def _():
                    fire_idx(u + 1, bm2)
            else:
                fire_idx(u + 1, bm2)
            drain_out(b)
            drain_idx(b)
            fire_gathers(u, b)

        fire_idx(0, 0)
        fire_idx(1, 1)
        fire_idx(2, 2)
        drain_idx(0)
        fire_gathers(0, 0)
        drain_idx(1)
        fire_gathers(1, 1)
        drain_gathers(0)
        fire_out(0, 0)
        fire_idx(3, 0)
        drain_idx(2)
        fire_gathers(2, 2)
        steady(3, 0, guard_idx=False)

        def step(s, carry):
            t = 3 * s + 4
            steady(t, 1, True)
            steady(t + 1, 2, True)
            steady(t + 2, 0, True)
            return carry

        lax.fori_loop(0, (groups - 4) // 3, step, 0)

        drain_gathers((groups - 2) % NBUF)
        fire_out(groups - 2, (groups - 2) % NBUF)
        drain_gathers((groups - 1) % NBUF)
        fire_out(groups - 1, (groups - 1) % NBUF)
        drain_out((groups - 3) % NBUF)
        drain_out((groups - 2) % NBUF)
        drain_out((groups - 1) % NBUF)

    return k


def kernel(inputs, table):
    b, h = inputs.shape
    vocab, dim = table.shape
    idx2d = inputs.astype(jnp.int32).reshape(-1, LANE)
    out = _build(idx2d.shape[0], vocab, dim)(idx2d, table)
    return out.reshape(b, h, dim)
